# baseline (speedup 1.0000x reference)
import functools

import jax
import jax.numpy as jnp
from jax import lax
from jax.experimental import pallas as pl
from jax.experimental.pallas import tpu as pltpu
from jax.experimental.pallas import tpu_sc as plsc

ROWS, COLS = 32768, 4096
LANES = 16
NUM_CORES = 2
NUM_SUBCORES = 16
NW = NUM_CORES * NUM_SUBCORES
SC_ROWS = 14336
TC_ROWS = ROWS - SC_ROWS
ROWS_PER_W = SC_ROWS // NW
R = 4
CHUNKS = ROWS_PER_W // R
VPR = COLS // LANES
UNROLL = 8

BR = 256
LB = 128


def _rev_body(in_hbm, out_hbm, ib0, ib1, ob0, ob1, ls0, ls1, ss0, ss1):
    ibs, obs = (ib0, ib1), (ob0, ob1)
    lss, sss = (ls0, ls1), (ss0, ss1)
    wid = lax.axis_index("s") * NUM_CORES + lax.axis_index("c")
    row0 = wid * ROWS_PER_W

    def load(g, b):
        pltpu.make_async_copy(
            in_hbm.at[pl.ds(row0 + g * R, R)], ibs[b], lss[b]).start()

    def wait_load(b):
        pltpu.make_async_copy(
            in_hbm.at[pl.ds(row0, R)], ibs[b], lss[b]).wait()

    def store(g, b):
        pltpu.make_async_copy(
            obs[b], out_hbm.at[pl.ds(row0 + g * R, R)], sss[b]).start()

    def wait_store(b):
        pltpu.make_async_copy(
            obs[b], out_hbm.at[pl.ds(row0, R)], sss[b]).wait()

    def compute(b):
        ibuf, obuf = ibs[b], obs[b]
        for r in range(R):
            @plsc.parallel_loop(0, VPR, 1, unroll=UNROLL)
            def _(k, r=r, ibuf=ibuf, obuf=obuf):
                v = ibuf[r, pl.ds(k * LANES, LANES)]
                obuf[r, pl.ds(COLS - LANES - k * LANES, LANES)] = (
                    lax.rev(v, (0,)))

    load(0, 0)

    def outer(gg, carry):
        wait_load(0)
        load(2 * gg + 1, 1)
        @pl.when(gg >= 1)
        def _():
            wait_store(0)
        compute(0)
        store(2 * gg, 0)

        wait_load(1)
        @pl.when(gg <= CHUNKS // 2 - 2)
        def _():
            load(2 * gg + 2, 0)
        @pl.when(gg >= 1)
        def _():
            wait_store(1)
        compute(1)
        store(2 * gg + 1, 1)
        return carry

    lax.fori_loop(0, CHUNKS // 2, outer, 0)
    wait_store(0)
    wait_store(1)


_sc_rev = functools.partial(
    pl.kernel,
    out_type=jax.ShapeDtypeStruct((SC_ROWS, COLS), jnp.float32),
    mesh=plsc.VectorSubcoreMesh(
        core_axis_name="c", subcore_axis_name="s",
        num_cores=NUM_CORES, num_subcores=NUM_SUBCORES),
    scratch_types=[
        pltpu.VMEM((R, COLS), jnp.float32),
        pltpu.VMEM((R, COLS), jnp.float32),
        pltpu.VMEM((R, COLS), jnp.float32),
        pltpu.VMEM((R, COLS), jnp.float32),
        pltpu.SemaphoreType.DMA,
        pltpu.SemaphoreType.DMA,
        pltpu.SemaphoreType.DMA,
        pltpu.SemaphoreType.DMA,
    ],
)(_rev_body)


def _tc_body(in_ref, out_ref):
    ridx = LB - 1 - lax.broadcasted_iota(jnp.int32, (BR, LB), 1)
    for c in range(COLS // LB):
        x = in_ref[:, pl.ds((COLS // LB - 1 - c) * LB, LB)]
        out_ref[:, pl.ds(c * LB, LB)] = jnp.take_along_axis(
            x, ridx, axis=1, mode="promise_in_bounds")


_tc_rev = pl.pallas_call(
    _tc_body,
    grid=(TC_ROWS // BR,),
    in_specs=[pl.BlockSpec((BR, COLS), lambda i: (i + SC_ROWS // BR, 0))],
    out_specs=pl.BlockSpec((BR, COLS), lambda i: (i, 0)),
    out_shape=jax.ShapeDtypeStruct((TC_ROWS, COLS), jnp.float32),
)


def kernel(inputs):
    sc_out = _sc_rev(inputs)
    tc_out = _tc_rev(inputs)
    return jnp.concatenate([sc_out, tc_out], axis=0)


# speedup vs baseline: 1.8105x; 1.8105x over previous
import functools

import jax
import jax.numpy as jnp
from jax import lax
from jax.experimental import pallas as pl
from jax.experimental.pallas import tpu as pltpu
from jax.experimental.pallas import tpu_sc as plsc

ROWS, COLS = 32768, 4096
LANES = 16
NUM_CORES = 2
NUM_SUBCORES = 16
NW = NUM_CORES * NUM_SUBCORES
ROWS_PER_W = ROWS // NW
R = 2
D = 4
CHUNKS = ROWS_PER_W // R
VPR = COLS // LANES
UNROLL = 8


def _rev_body(in_hbm, out_hbm, *refs):
    ibs, obs = refs[0:D], refs[D:2 * D]
    lss, sss = refs[2 * D:3 * D], refs[3 * D:4 * D]
    wid = lax.axis_index("s") * NUM_CORES + lax.axis_index("c")
    row0 = wid * ROWS_PER_W

    def load(g, b):
        pltpu.make_async_copy(
            in_hbm.at[pl.ds(row0 + g * R, R)], ibs[b], lss[b]).start()

    def wait_load(b):
        pltpu.make_async_copy(
            in_hbm.at[pl.ds(row0, R)], ibs[b], lss[b]).wait()

    def store(g, b):
        pltpu.make_async_copy(
            obs[b], out_hbm.at[pl.ds(row0 + g * R, R)], sss[b]).start()

    def wait_store(b):
        pltpu.make_async_copy(
            obs[b], out_hbm.at[pl.ds(row0, R)], sss[b]).wait()

    def compute(b):
        ibuf, obuf = ibs[b], obs[b]
        for r in range(R):
            @plsc.parallel_loop(0, VPR, 1, unroll=UNROLL)
            def _(k, r=r, ibuf=ibuf, obuf=obuf):
                v = ibuf[r, pl.ds(k * LANES, LANES)]
                obuf[r, pl.ds(COLS - LANES - k * LANES, LANES)] = (
                    lax.rev(v, (0,)))

    for b in range(D):
        load(b, b)

    def outer(gg, carry):
        for b in range(D):
            g = gg * D + b
            wait_load(b)
            @pl.when(gg >= 1)
            def _(b=b):
                wait_store(b)
            compute(b)
            store(g, b)
            @pl.when(gg <= CHUNKS // D - 2)
            def _(g=g, b=b):
                load(g + D, b)
        return carry

    lax.fori_loop(0, CHUNKS // D, outer, 0)
    for b in range(D):
        wait_store(b)


_rev_kernel = functools.partial(
    pl.kernel,
    out_type=jax.ShapeDtypeStruct((ROWS, COLS), jnp.float32),
    mesh=plsc.VectorSubcoreMesh(
        core_axis_name="c", subcore_axis_name="s",
        num_cores=NUM_CORES, num_subcores=NUM_SUBCORES),
    scratch_types=(
        [pltpu.VMEM((R, COLS), jnp.float32)] * (2 * D)
        + [pltpu.SemaphoreType.DMA] * (2 * D)
    ),
)(_rev_body)


def kernel(inputs):
    return _rev_kernel(inputs)
